# baseline (speedup 1.0000x reference)
import functools

import jax
import jax.numpy as jnp
import numpy as np
from jax import lax
from jax.experimental import pallas as pl
from jax.experimental.pallas import tpu as pltpu
from jax.experimental.pallas import tpu_sc as plsc

N = 10000
E = 320000
D = 128
C = 16

NC = 2
NS = 16
EPT = 20480
EPAD = EPT * NS
NP = 10240
RPT = NP // NS


def _agg_body(width, CH, EB, NSTG, bf16_table, xcat, srcs, dsts, ews, out,
              acc, ssrc, sdst, sew, bufg, bufs,
              semg0, semg1, sems0, sems1, semt0, semt1):
    NCH = NSTG * EB
    c = lax.axis_index("c")
    s = lax.axis_index("s")
    nfg = width // 16

    semg = [semg0, semg1]
    sems = [sems0, sems1]
    semt = [semt0, semt1]

    zvec = jnp.zeros((16,), jnp.float32)

    def zrow(r, carry):
        for j in range(nfg):
            bufs[0, r, pl.ds(j * 16, 16)] = zvec
        return carry

    lax.fori_loop(0, CH, zrow, 0)
    row0 = s * RPT
    for k in range(RPT // CH):
        pltpu.async_copy(bufs.at[0], acc.at[pl.ds(row0 + k * CH, CH)],
                         semt[0])
    for k in range(RPT // CH):
        pltpu.make_async_copy(bufs.at[0], acc.at[pl.ds(row0 + k * CH, CH)],
                              semt[0]).wait()
    plsc.subcore_barrier()

    tile_base = (c * NS + s) * NCH

    def stage_start(st, p):
        base = tile_base + st * EB
        pltpu.async_copy(srcs.at[pl.ds(base, EB)], ssrc.at[p], semt[p])
        pltpu.async_copy(dsts.at[pl.ds(base, EB)], sdst.at[p], semt[p])
        pltpu.async_copy(ews.at[pl.ds(base, EB)], sew.at[p], semt[p])

    def stage_wait(st, p):
        base = tile_base + st * EB
        pltpu.make_async_copy(srcs.at[pl.ds(base, EB)], ssrc.at[p],
                              semt[p]).wait()
        pltpu.make_async_copy(dsts.at[pl.ds(base, EB)], sdst.at[p],
                              semt[p]).wait()
        pltpu.make_async_copy(ews.at[pl.ds(base, EB)], sew.at[p],
                              semt[p]).wait()

    def gather_start(p, r, b):
        pltpu.async_copy(xcat.at[ssrc.at[p].at[r]], bufg.at[b], semg[b])

    def gather_wait(p, r, b):
        pltpu.make_async_copy(xcat.at[ssrc.at[p].at[r]], bufg.at[b],
                              semg[b]).wait()

    def scatter_start(p, r, b):
        pltpu.async_copy(bufs.at[b], acc.at[sdst.at[p].at[r]], sems[b],
                         add=True)

    def scatter_wait(b):
        pltpu.make_async_copy(bufs.at[b], acc.at[sdst.at[0].at[0]],
                              sems[b]).wait()

    def scale(p, r, b):
        def sgrp(g, cc):
            wv = sew[p, r, pl.ds(g * 16, 16)]
            for k in range(16):
                rr = g * 16 + k
                w = wv[k]
                if bf16_table:
                    wb = jnp.full((16,), w, jnp.float32)
                    wp = plsc.pack(wb, wb, format=plsc.PackFormat.INTERLEAVED)
                    for g2 in range(width // 32):
                        v = bufg[b, rr, pl.ds(g2 * 32, 32)]
                        prod = v * wp
                        av, bv = plsc.unpack(
                            prod, format=plsc.PackFormat.INTERLEAVED)
                        bufs[b, rr, pl.ds(g2 * 32, 16)] = av
                        bufs[b, rr, pl.ds(g2 * 32 + 16, 16)] = bv
                else:
                    for j in range(nfg):
                        sl = pl.ds(j * 16, 16)
                        bufs[b, rr, sl] = bufg[b, rr, sl] * w
            return cc

        lax.fori_loop(0, CH // 16, sgrp, 0)

    def chunk_body(st, p, r, b, first_pair):
        gather_wait(p, r, b)
        if not first_pair:
            scatter_wait(b)
        else:
            @pl.when(st > 0)
            def _():
                scatter_wait(b)
        scale(p, r, b)
        scatter_start(p, r, b)
        nr = r + 2

        @pl.when(nr < EB)
        def _():
            gather_start(p, nr, b)

        @pl.when((nr >= EB) & (st < NSTG - 1))
        def _():
            @pl.when(nr == EB)
            def _w():
                stage_wait(st + 1, 1 - p)
            gather_start(1 - p, nr - EB, b)

    def stage_body(st, p):
        for b in range(2):
            chunk_body(st, p, b, b, True)
        @pl.when(st < NSTG - 1)
        def _():
            stage_start(st + 1, 1 - p)

        def pair(i, carry):
            for b in range(2):
                chunk_body(st, p, 2 * i + b, b, False)
            return carry

        lax.fori_loop(1, EB // 2, pair, 0)

    stage_start(0, 0)
    stage_wait(0, 0)
    for b in range(2):
        gather_start(0, b, b)

    def stage_pair(k, carry):
        stage_body(2 * k, 0)
        stage_body(2 * k + 1, 1)
        return carry

    lax.fori_loop(0, NSTG // 2, stage_pair, 0)

    for b in range(2):
        scatter_wait(b)

    plsc.subcore_barrier()
    pltpu.sync_copy(acc.at[pl.ds(s * RPT, RPT)],
                    out.at[pl.ds(c * NP + s * RPT, RPT)])


def _make_agg(width, CH, EB, NSTG, bf16_table=False):
    assert CH * EB * NSTG == EPT and NSTG % 2 == 0 and EB % 2 == 0
    gdt = jnp.bfloat16 if bf16_table else jnp.float32
    mesh = plsc.VectorSubcoreMesh(core_axis_name="c", subcore_axis_name="s",
                                  num_cores=NC, num_subcores=NS)
    return pl.kernel(
        functools.partial(_agg_body, width, CH, EB, NSTG, bf16_table),
        out_type=jax.ShapeDtypeStruct((2 * NP, width), jnp.float32),
        mesh=mesh,
        compiler_params=pltpu.CompilerParams(
            use_tc_tiling_on_sc=False,
            needs_layout_passes=not bf16_table),
        scratch_types=[
            pltpu.VMEM_SHARED((NP, width), jnp.float32),
            pltpu.VMEM((2, EB, CH), jnp.int32),
            pltpu.VMEM((2, EB, CH), jnp.int32),
            pltpu.VMEM((2, EB, CH), jnp.float32),
            pltpu.VMEM((2, CH, width), gdt),
            pltpu.VMEM((2, CH, width), jnp.float32),
            pltpu.SemaphoreType.DMA,
            pltpu.SemaphoreType.DMA,
            pltpu.SemaphoreType.DMA,
            pltpu.SemaphoreType.DMA,
            pltpu.SemaphoreType.DMA,
            pltpu.SemaphoreType.DMA,
        ],
        name=f"mgcn_agg{width}",
    )


CH128, EB128, NSTG128 = 64, 20, 16
CH16, EB16, NSTG16 = 128, 20, 8
_agg128 = _make_agg(D, CH128, EB128, NSTG128, bf16_table=True)
_agg16 = _make_agg(C, CH16, EB16, NSTG16)

_PERM = np.concatenate(
    [np.concatenate([np.arange(g, g + 32, 2), np.arange(g + 1, g + 32, 2)])
     for g in range(0, D, 32)])


def _tc_body(x_ref, w0_ref, w1_ref, o_ref):
    t = x_ref[...]
    h = jnp.maximum(jnp.dot(t, w0_ref[0], preferred_element_type=jnp.float32),
                    0.0)
    o_ref[...] = jnp.dot(h, w1_ref[0],
                         preferred_element_type=jnp.float32) * 0.5


_BLK = 1024


def _tc_transform(xagg, w0s, w1s):
    grid = (2 * NP // _BLK,)
    per = NP // _BLK
    return pl.pallas_call(
        _tc_body,
        grid=grid,
        in_specs=[
            pl.BlockSpec((_BLK, D), lambda p: (p, 0)),
            pl.BlockSpec((1, D, D), lambda p: (p // per, 0, 0)),
            pl.BlockSpec((1, D, C), lambda p: (p // per, 0, 0)),
        ],
        out_specs=pl.BlockSpec((_BLK, C), lambda p: (p, 0)),
        out_shape=jax.ShapeDtypeStruct((2 * NP, C), jnp.float32),
    )(xagg, w0s, w1s)


def _prep_edges(edge_index, edge_weight, branch):
    src = jnp.pad(edge_index[0], (0, EPAD - E)) + branch * NP
    dst = jnp.pad(edge_index[1], (0, EPAD - E))
    ew = jnp.pad(edge_weight, (0, EPAD - E))
    return src, dst, ew


def _layout(flat0, flat1, ch):
    nch = EPT // ch
    return jnp.concatenate([flat0.reshape(NS, nch, ch),
                            flat1.reshape(NS, nch, ch)]).reshape(-1, ch)


def kernel(x0, x1, edge_index0, edge_index1, edge_weight0, edge_weight1,
           W0_0, W1_0, W0_1, W1_1):
    s0, d0, w0 = _prep_edges(edge_index0, edge_weight0, 0)
    s1, d1, w1 = _prep_edges(edge_index1, edge_weight1, 1)
    srcs_a = _layout(s0, s1, CH128)
    dsts_a = _layout(d0, d1, CH128)
    ews_a = _layout(w0, w1, CH128)
    srcs_b = _layout(s0, s1, CH16)
    dsts_b = _layout(d0, d1, CH16)
    ews_b = _layout(w0, w1, CH16)

    xcat = jnp.zeros((2 * NP, D), jnp.bfloat16)
    xcat = xcat.at[:N].set(x0.astype(jnp.bfloat16))
    xcat = xcat.at[NP:NP + N].set(x1.astype(jnp.bfloat16))
    xagg = _agg128(xcat, srcs_a, dsts_a, ews_a)
    w0s = jnp.stack([W0_0, W0_1])[:, _PERM, :]
    y = _tc_transform(xagg, w0s,
                      jnp.stack([W1_0, W1_1]))
    parts = _agg16(y, srcs_b, dsts_b, ews_b)
    return parts[:N] + parts[NP:NP + N]


# speedup vs baseline: 1.0024x; 1.0024x over previous
import functools

import jax
import jax.numpy as jnp
import numpy as np
from jax import lax
from jax.experimental import pallas as pl
from jax.experimental.pallas import tpu as pltpu
from jax.experimental.pallas import tpu_sc as plsc

N = 10000
E = 320000
D = 128
C = 16

NC = 2
NS = 16
EPT = 20480
EPAD = EPT * NS
NP = 10240
RPT = NP // NS


def _agg_body(width, CH, EB, NSTG, bf16_table, xcat, srcs, dsts, ews, out,
              acc, ssrc, sdst, sew, bufg, bufs,
              semg0, semg1, sems0, sems1, semt0, semt1):
    NCH = NSTG * EB
    c = lax.axis_index("c")
    s = lax.axis_index("s")
    nfg = width // 16

    semg = [semg0, semg1]
    sems = [sems0, sems1]
    semt = [semt0, semt1]

    zvec = jnp.zeros((16,), jnp.float32)

    def zrow(r, carry):
        for j in range(nfg):
            bufs[0, r, pl.ds(j * 16, 16)] = zvec
        return carry

    lax.fori_loop(0, CH, zrow, 0)
    row0 = s * RPT
    for k in range(RPT // CH):
        pltpu.async_copy(bufs.at[0], acc.at[pl.ds(row0 + k * CH, CH)],
                         semt[0])
    for k in range(RPT // CH):
        pltpu.make_async_copy(bufs.at[0], acc.at[pl.ds(row0 + k * CH, CH)],
                              semt[0]).wait()
    plsc.subcore_barrier()

    tile_base = (c * NS + s) * NCH

    def stage_start(st, p):
        base = tile_base + st * EB
        pltpu.async_copy(srcs.at[pl.ds(base, EB)], ssrc.at[p], semt[p])
        pltpu.async_copy(dsts.at[pl.ds(base, EB)], sdst.at[p], semt[p])
        pltpu.async_copy(ews.at[pl.ds(base, EB)], sew.at[p], semt[p])

    def stage_wait(st, p):
        base = tile_base + st * EB
        pltpu.make_async_copy(srcs.at[pl.ds(base, EB)], ssrc.at[p],
                              semt[p]).wait()
        pltpu.make_async_copy(dsts.at[pl.ds(base, EB)], sdst.at[p],
                              semt[p]).wait()
        pltpu.make_async_copy(ews.at[pl.ds(base, EB)], sew.at[p],
                              semt[p]).wait()

    def gather_start(p, r, b):
        pltpu.async_copy(xcat.at[ssrc.at[p].at[r]], bufg.at[b], semg[b])

    def gather_wait(p, r, b):
        pltpu.make_async_copy(xcat.at[ssrc.at[p].at[r]], bufg.at[b],
                              semg[b]).wait()

    def scatter_start(p, r, b):
        pltpu.async_copy(bufs.at[b], acc.at[sdst.at[p].at[r]], sems[b],
                         add=True)

    def scatter_wait(b):
        pltpu.make_async_copy(bufs.at[b], acc.at[sdst.at[0].at[0]],
                              sems[b]).wait()

    def scale(p, r, b):
        def sgrp(g, cc):
            wv = sew[p, r, pl.ds(g * 16, 16)]
            for k in range(16):
                rr = g * 16 + k
                w = wv[k]
                if bf16_table:
                    for g2 in range(width // 32):
                        v = bufg[b, rr, pl.ds(g2 * 32, 32)]
                        av, bv = plsc.unpack(
                            v, format=plsc.PackFormat.INTERLEAVED)
                        bufs[b, rr, pl.ds(g2 * 32, 16)] = av * w
                        bufs[b, rr, pl.ds(g2 * 32 + 16, 16)] = bv * w
                else:
                    for j in range(nfg):
                        sl = pl.ds(j * 16, 16)
                        bufs[b, rr, sl] = bufg[b, rr, sl] * w
            return cc

        lax.fori_loop(0, CH // 16, sgrp, 0)

    def chunk_body(st, p, r, b, first_pair):
        gather_wait(p, r, b)
        if not first_pair:
            scatter_wait(b)
        else:
            @pl.when(st > 0)
            def _():
                scatter_wait(b)
        scale(p, r, b)
        scatter_start(p, r, b)
        nr = r + 2

        @pl.when(nr < EB)
        def _():
            gather_start(p, nr, b)

        @pl.when((nr >= EB) & (st < NSTG - 1))
        def _():
            @pl.when(nr == EB)
            def _w():
                stage_wait(st + 1, 1 - p)
            gather_start(1 - p, nr - EB, b)

    def stage_body(st, p):
        for b in range(2):
            chunk_body(st, p, b, b, True)
        @pl.when(st < NSTG - 1)
        def _():
            stage_start(st + 1, 1 - p)

        def pair(i, carry):
            for b in range(2):
                chunk_body(st, p, 2 * i + b, b, False)
            return carry

        lax.fori_loop(1, EB // 2, pair, 0)

    stage_start(0, 0)
    stage_wait(0, 0)
    for b in range(2):
        gather_start(0, b, b)

    def stage_pair(k, carry):
        stage_body(2 * k, 0)
        stage_body(2 * k + 1, 1)
        return carry

    lax.fori_loop(0, NSTG // 2, stage_pair, 0)

    for b in range(2):
        scatter_wait(b)

    plsc.subcore_barrier()
    pltpu.sync_copy(acc.at[pl.ds(s * RPT, RPT)],
                    out.at[pl.ds(c * NP + s * RPT, RPT)])


def _make_agg(width, CH, EB, NSTG, bf16_table=False):
    assert CH * EB * NSTG == EPT and NSTG % 2 == 0 and EB % 2 == 0
    gdt = jnp.bfloat16 if bf16_table else jnp.float32
    mesh = plsc.VectorSubcoreMesh(core_axis_name="c", subcore_axis_name="s",
                                  num_cores=NC, num_subcores=NS)
    return pl.kernel(
        functools.partial(_agg_body, width, CH, EB, NSTG, bf16_table),
        out_type=jax.ShapeDtypeStruct((2 * NP, width), jnp.float32),
        mesh=mesh,
        compiler_params=pltpu.CompilerParams(
            use_tc_tiling_on_sc=False,
            needs_layout_passes=not bf16_table),
        scratch_types=[
            pltpu.VMEM_SHARED((NP, width), jnp.float32),
            pltpu.VMEM((2, EB, CH), jnp.int32),
            pltpu.VMEM((2, EB, CH), jnp.int32),
            pltpu.VMEM((2, EB, CH), jnp.float32),
            pltpu.VMEM((2, CH, width), gdt),
            pltpu.VMEM((2, CH, width), jnp.float32),
            pltpu.SemaphoreType.DMA,
            pltpu.SemaphoreType.DMA,
            pltpu.SemaphoreType.DMA,
            pltpu.SemaphoreType.DMA,
            pltpu.SemaphoreType.DMA,
            pltpu.SemaphoreType.DMA,
        ],
        name=f"mgcn_agg{width}",
    )


CH128, EB128, NSTG128 = 64, 20, 16
CH16, EB16, NSTG16 = 128, 20, 8
_agg128 = _make_agg(D, CH128, EB128, NSTG128, bf16_table=True)
_agg16 = _make_agg(C, CH16, EB16, NSTG16)

_PERM = np.concatenate(
    [np.concatenate([np.arange(g, g + 32, 2), np.arange(g + 1, g + 32, 2)])
     for g in range(0, D, 32)])


def _tc_body(x_ref, w0_ref, w1_ref, o_ref):
    t = x_ref[...]
    h = jnp.maximum(jnp.dot(t, w0_ref[0], preferred_element_type=jnp.float32),
                    0.0)
    o_ref[...] = jnp.dot(h, w1_ref[0],
                         preferred_element_type=jnp.float32) * 0.5


_BLK = 1024


def _tc_transform(xagg, w0s, w1s):
    grid = (2 * NP // _BLK,)
    per = NP // _BLK
    return pl.pallas_call(
        _tc_body,
        grid=grid,
        in_specs=[
            pl.BlockSpec((_BLK, D), lambda p: (p, 0)),
            pl.BlockSpec((1, D, D), lambda p: (p // per, 0, 0)),
            pl.BlockSpec((1, D, C), lambda p: (p // per, 0, 0)),
        ],
        out_specs=pl.BlockSpec((_BLK, C), lambda p: (p, 0)),
        out_shape=jax.ShapeDtypeStruct((2 * NP, C), jnp.float32),
    )(xagg, w0s, w1s)


def _prep_edges(edge_index, edge_weight, branch):
    src = jnp.pad(edge_index[0], (0, EPAD - E)) + branch * NP
    dst = jnp.pad(edge_index[1], (0, EPAD - E))
    ew = jnp.pad(edge_weight, (0, EPAD - E))
    return src, dst, ew


def _layout(flat0, flat1, ch):
    nch = EPT // ch
    return jnp.concatenate([flat0.reshape(NS, nch, ch),
                            flat1.reshape(NS, nch, ch)]).reshape(-1, ch)


def kernel(x0, x1, edge_index0, edge_index1, edge_weight0, edge_weight1,
           W0_0, W1_0, W0_1, W1_1):
    s0, d0, w0 = _prep_edges(edge_index0, edge_weight0, 0)
    s1, d1, w1 = _prep_edges(edge_index1, edge_weight1, 1)
    srcs_a = _layout(s0, s1, CH128)
    dsts_a = _layout(d0, d1, CH128)
    ews_a = _layout(w0, w1, CH128)
    srcs_b = _layout(s0, s1, CH16)
    dsts_b = _layout(d0, d1, CH16)
    ews_b = _layout(w0, w1, CH16)

    xcat = jnp.zeros((2 * NP, D), jnp.bfloat16)
    xcat = xcat.at[:N].set(x0.astype(jnp.bfloat16))
    xcat = xcat.at[NP:NP + N].set(x1.astype(jnp.bfloat16))
    xagg = _agg128(xcat, srcs_a, dsts_a, ews_a)
    w0s = jnp.stack([W0_0, W0_1])[:, _PERM, :]
    y = _tc_transform(xagg, w0s,
                      jnp.stack([W1_0, W1_1]))
    parts = _agg16(y, srcs_b, dsts_b, ews_b)
    return parts[:N] + parts[NP:NP + N]
